# baseline (speedup 1.0000x reference)
import jax
import jax.numpy as jnp
import numpy as np
from jax.experimental import pallas as pl
from jax.experimental.pallas import tpu as pltpu
from jax.experimental.shard_map import shard_map
from jax.sharding import Mesh, PartitionSpec


def _unembed_tile(x_ref, w_ref, o_ref):
    o_ref[...] = jnp.dot(
        x_ref[...], w_ref[...], preferred_element_type=jnp.float32
    )


def _unembed_block(x2d, w):
    rows, d_emb = x2d.shape
    d_vocab = w.shape[1]
    tm = min(4096, rows)
    tn = 512
    grid = (pl.cdiv(rows, tm), pl.cdiv(d_vocab, tn))
    return pl.pallas_call(
        _unembed_tile,
        grid=grid,
        in_specs=[
            pl.BlockSpec((tm, d_emb), lambda i, j: (i, 0)),
            pl.BlockSpec((d_emb, tn), lambda i, j: (0, j)),
        ],
        out_specs=pl.BlockSpec((tm, tn), lambda i, j: (i, j)),
        out_shape=jax.ShapeDtypeStruct((rows, d_vocab), jnp.float32),
        compiler_params=pltpu.CompilerParams(
            dimension_semantics=("parallel", "parallel"),
            vmem_limit_bytes=60 * 1024 * 1024,
        ),
    )(x2d, w)


def kernel(x, w_u):
    b, p, d_emb = x.shape
    d_emb_w, d_vocab = w_u.shape
    assert d_emb == d_emb_w

    rows = b * p
    x2d = x.reshape(rows, d_emb).astype(jnp.bfloat16)
    w16 = w_u.astype(jnp.bfloat16)

    devs = jax.devices()
    if len(devs) >= 2 and d_vocab % 256 == 0:
        mesh = Mesh(np.asarray(devs[:2]), ("v",))
        x2d = jax.lax.with_sharding_constraint(
            x2d, jax.sharding.NamedSharding(mesh, PartitionSpec())
        )
        w16 = jax.lax.with_sharding_constraint(
            w16, jax.sharding.NamedSharding(mesh, PartitionSpec(None, "v"))
        )
        out2d = shard_map(
            _unembed_block,
            mesh=mesh,
            in_specs=(PartitionSpec(None, None), PartitionSpec(None, "v")),
            out_specs=PartitionSpec(None, "v"),
            check_rep=False,
        )(x2d, w16)
    else:
        out2d = _unembed_block(x2d, w16)

    return out2d.reshape(b, p, d_vocab)


# speedup vs baseline: 1.0485x; 1.0485x over previous
import jax
import jax.numpy as jnp
import numpy as np
from jax.experimental import pallas as pl
from jax.experimental.pallas import tpu as pltpu
from jax.experimental.shard_map import shard_map
from jax.sharding import Mesh, PartitionSpec


def _unembed_tile(x_ref, w_ref, o_ref):
    o_ref[...] = jnp.dot(
        x_ref[...],
        w_ref[...].astype(jnp.bfloat16),
        preferred_element_type=jnp.float32,
    )


def _unembed_block(x2d, w):
    rows, d_emb = x2d.shape
    d_vocab = w.shape[1]
    tm = min(4096, rows)
    tn = 256
    grid = (pl.cdiv(rows, tm), pl.cdiv(d_vocab, tn))
    return pl.pallas_call(
        _unembed_tile,
        grid=grid,
        in_specs=[
            pl.BlockSpec((tm, d_emb), lambda i, j: (i, 0)),
            pl.BlockSpec((d_emb, tn), lambda i, j: (0, j)),
        ],
        out_specs=pl.BlockSpec((tm, tn), lambda i, j: (i, j)),
        out_shape=jax.ShapeDtypeStruct((rows, d_vocab), jnp.float32),
        compiler_params=pltpu.CompilerParams(
            dimension_semantics=("parallel", "parallel"),
            vmem_limit_bytes=60 * 1024 * 1024,
        ),
    )(x2d, w)


def kernel(x, w_u):
    b, p, d_emb = x.shape
    d_emb_w, d_vocab = w_u.shape
    assert d_emb == d_emb_w

    rows = b * p

    devs = jax.devices()
    if len(devs) >= 2 and d_vocab % 256 == 0:
        mesh = Mesh(np.asarray(devs[:2]), ("v",))
        x = jax.lax.with_sharding_constraint(
            x, jax.sharding.NamedSharding(mesh, PartitionSpec())
        )
        w_u = jax.lax.with_sharding_constraint(
            w_u, jax.sharding.NamedSharding(mesh, PartitionSpec(None, "v"))
        )
        x2d = x.reshape(rows, d_emb).astype(jnp.bfloat16)
        out2d = shard_map(
            _unembed_block,
            mesh=mesh,
            in_specs=(PartitionSpec(None, None), PartitionSpec(None, "v")),
            out_specs=PartitionSpec(None, "v"),
            check_rep=False,
        )(x2d, w_u)
    else:
        out2d = _unembed_block(
            x.reshape(rows, d_emb).astype(jnp.bfloat16), w_u
        )

    return out2d.reshape(b, p, d_vocab)


# speedup vs baseline: 1.0751x; 1.0254x over previous
import jax
import jax.numpy as jnp
import numpy as np
from jax.experimental import pallas as pl
from jax.experimental.pallas import tpu as pltpu
from jax.experimental.shard_map import shard_map
from jax.sharding import Mesh, PartitionSpec


def _unembed_tile(x_ref, w_ref, o_ref):
    o_ref[...] = jnp.dot(
        x_ref[...], w_ref[...], preferred_element_type=jnp.float32
    )


def _unembed_block(x2d, w):
    rows, d_emb = x2d.shape
    d_vocab = w.shape[1]
    tm = min(2048, rows)
    tn = 512
    grid = (pl.cdiv(rows, tm), pl.cdiv(d_vocab, tn))
    return pl.pallas_call(
        _unembed_tile,
        grid=grid,
        in_specs=[
            pl.BlockSpec((tm, d_emb), lambda i, j: (i, 0)),
            pl.BlockSpec((d_emb, tn), lambda i, j: (0, j)),
        ],
        out_specs=pl.BlockSpec((tm, tn), lambda i, j: (i, j)),
        out_shape=jax.ShapeDtypeStruct((rows, d_vocab), jnp.float32),
        compiler_params=pltpu.CompilerParams(
            dimension_semantics=("parallel", "parallel"),
            vmem_limit_bytes=60 * 1024 * 1024,
        ),
    )(x2d, w)


def kernel(x, w_u):
    b, p, d_emb = x.shape
    d_emb_w, d_vocab = w_u.shape
    assert d_emb == d_emb_w

    rows = b * p

    devs = jax.devices()
    if len(devs) >= 2 and d_vocab % 256 == 0:
        mesh = Mesh(np.asarray(devs[:2]), ("v",))
        x = jax.lax.with_sharding_constraint(
            x, jax.sharding.NamedSharding(mesh, PartitionSpec())
        )
        w_u = jax.lax.with_sharding_constraint(
            w_u, jax.sharding.NamedSharding(mesh, PartitionSpec(None, "v"))
        )
        x2d = x.reshape(rows, d_emb)
        out2d = shard_map(
            _unembed_block,
            mesh=mesh,
            in_specs=(PartitionSpec(None, None), PartitionSpec(None, "v")),
            out_specs=PartitionSpec(None, "v"),
            check_rep=False,
        )(x2d, w_u)
    else:
        out2d = _unembed_block(x.reshape(rows, d_emb), w_u)

    return out2d.reshape(b, p, d_vocab)
